# baseline (speedup 1.0000x reference)
import functools
import math

import jax
import jax.numpy as jnp
import numpy as np
from jax.experimental import pallas as pl
from jax.experimental.pallas import tpu as pltpu

B, A, NBH = 4, 64, 48
CUTOFF = 10.0
N_GAUSS = 50
NF = 128
NCB = 32
DIRS = 4
LMAX = 4
N_SCHNET = 2
N_DIR = 2 * LMAX
NLAYER = 1 + N_DIR
RWS = A * NBH
MAX_Z = 100

_OFFS = np.linspace(0.0, CUTOFF, N_GAUSS).astype(np.float32)
_GCOEF = np.float32(-0.5 / (_OFFS[1] - _OFFS[0]) ** 2)
_LN2 = np.float32(math.log(2.0))

_R4 = np.kron(np.eye(NCB, dtype=np.float32), np.ones((1, 4), np.float32))
_R3 = np.kron(np.eye(NCB, dtype=np.float32), np.ones((1, 3), np.float32))
_T3 = np.zeros((8, 3 * NCB), np.float32)
for _d in range(3):
    _T3[_d, _d::3] = 1.0

_PREC = jax.lax.Precision.DEFAULT
_BF = jnp.bfloat16


def _mm(a, b, prec=_PREC, out=_BF):
    res = jax.lax.dot_general(
        a, b, (((a.ndim - 1,), (0,)), ((), ())),
        precision=prec, preferred_element_type=jnp.float32)
    return res.astype(out)


_INV_LN2 = float(1.0 / math.log(2.0))
_LN2F = float(math.log(2.0))


def _ssp(x):
    xf = x.astype(jnp.float32)
    t = jnp.exp2(-jnp.abs(xf) * _INV_LN2)
    out = jnp.maximum(xf, 0.0) + _LN2F * (jnp.log2(1.0 + t) - 1.0)
    return out.astype(x.dtype)


def _body(names, *refs):
    r = dict(zip(names, refs))
    i = pl.program_id(1)

    def w(name, squeeze=False):
        ref = r[name]
        return ref[0] if squeeze else ref[...]

    oh_s = r['oh_s']
    geo_s = r['geo_s']
    g_s = r['g_s']
    x_s = r['x_s']
    p_s = r['p_s']

    def filters(f1w, f1b, f2w, f2b, g, cut):
        W = _mm(_ssp(_mm(g, f1w) + f1b), f2w) + f2b
        return W * cut

    def add_per_atom(v, y):
        f = v.shape[-1]
        return (v.reshape(A, NBH, f) + y[:, None, :]).reshape(RWS, f)

    def mul_per_atom(v, y):
        f = v.shape[-1]
        return (v.reshape(A, NBH, f) * y[:, None, :]).reshape(RWS, f)

    def nsum(v):
        return jnp.sum(v.reshape(A, NBH, v.shape[-1]), axis=1)

    def ft_and_heads(x, oh, g, cut, p1w, p1b, p2w, p2b, inw, ftw, ftb,
                     a1w, a1b, a2w, a2b, e1w, e1b, e2w, e2b,
                     fw1, fb1, fw2, fb2):
        W = filters(fw1, fb1, fw2, fb2, g, cut)
        y = _mm(x.astype(_BF), inw)
        v = _ssp(_mm(mul_per_atom(_mm(oh, y), y) * W, ftw) + ftb)
        vi = (_mm(_ssp(_mm(nsum(v), a1w) + a1b), a2w, out=jnp.float32)
              + a2b.astype(jnp.float32))
        vij = _mm(_ssp(_mm(v, p1w) + p1b), p2w) + p2b
        vik = _mm(_ssp(_mm(v, e1w) + e1b), e2w) + e2b
        return vi, vij, vik

    @pl.when(i == 0)
    def _prologue():
        az = r['az'][0, 0]
        zidx = jax.lax.broadcasted_iota(jnp.int32, (A, MAX_Z), 1)
        ohz = (az[:, None] == zidx).astype(jnp.float32)
        x0 = _mm(ohz, w('emb'), jax.lax.Precision.HIGHEST,
                 out=jnp.float32)
        r['x0_o'][0] = x0

        nb = r['nbr'][0]
        jidx = jax.lax.broadcasted_iota(jnp.int32, (A, NBH, A), 2)
        ohf = (nb[:, :, None] == jidx).astype(jnp.float32).reshape(RWS, A)
        oh = ohf.astype(_BF)
        oh_s[...] = oh

        pos = r['pos'][0]
        pos_j = _mm(ohf, pos, out=jnp.float32)
        d = (pos_j.reshape(A, NBH, 3) - pos[:, None, :]).reshape(RWS, 3)
        s = jnp.sum(d * d, axis=1, keepdims=True)
        rij = jnp.sqrt(jnp.where(s > 0, s, 1.0)) * (s > 0).astype(jnp.float32)
        cos = d / jnp.where(rij > 0, rij, 1.0)
        cut = (rij <= CUTOFF).astype(jnp.float32)
        geo_s[...] = jnp.concatenate(
            [cos, cut, jnp.zeros((RWS, 4), jnp.float32)], axis=1).astype(_BF)
        g = jnp.exp(_GCOEF * (rij - w('offs')) ** 2).astype(_BF)
        g_s[...] = g
        cutb = cut.astype(_BF)

        x = x0
        for k in range(N_SCHNET):
            W = filters(w('s_f1w')[k], w('s_f1b')[k], w('s_f2w')[k],
                        w('s_f2b')[k], g, cutb)
            y_j = _mm(oh, _mm(x.astype(_BF), w('s_inw')[k]))
            ysum = nsum(y_j * W)
            y2 = _ssp(_mm(ysum, w('s_ow')[k]) + w('s_ob')[k])
            x = (x + _mm(y2, w('s_dw')[k], out=jnp.float32)
                 + w('s_db')[k].astype(jnp.float32))

        vi, vij, vik = ft_and_heads(
            x, oh, g, cutb,
            w('f_p1w'), w('f_p1b'), w('f_p2w'), w('f_p2b'),
            w('f_inw'), w('f_ftw'), w('f_ftb'),
            w('f_a1w'), w('f_a1b'), w('f_a2w'), w('f_a2b'),
            w('f_e1w'), w('f_e1b'), w('f_e2w'), w('f_e2b'),
            w('f_f1w'), w('f_f1b'), w('f_f2w'), w('f_f2b'))
        vik32 = nsum(vik)
        v32 = add_per_atom(vij + _mm(oh, vik32), vik32)
        P = _mm(v32, w('r4c'))
        p_s[...] = P
        r['xij_o'][0, 0] = P.astype(jnp.float32).reshape(A, NBH, NF)
        x_s[...] = x + vi

    @pl.when(i > 0)
    def _direction_layer():
        oh = oh_s[...]
        geo = geo_s[...]
        g = g_s[...]
        cut = geo[:, 3:4]
        x = x_s[...]

        vi, vij, vik = ft_and_heads(
            x, oh, g, cut,
            w('i_p1w', True), w('i_p1b', True), w('i_p2w', True),
            w('i_p2b', True), w('i_inw', True), w('i_ftw', True),
            w('i_ftb', True), w('i_a1w', True), w('i_a1b', True),
            w('i_a2w', True), w('i_a2b', True), w('i_e1w', True),
            w('i_e1b', True), w('i_e2w', True), w('i_e2b', True),
            w('i_f1w', True), w('i_f1b', True), w('i_f2w', True),
            w('i_f2b', True))

        cosx96 = _mm(geo, w('t3c'))
        vik96 = _mm(vik, w('r3c')) * cosx96
        Vik96 = nsum(vik96)
        Vjl = _mm(_mm(oh, Vik96), w('i_g2', True))
        Vik = _mm(Vik96, w('i_g1', True))
        cmx = _mm(geo, w('i_pmt', True))
        V = add_per_atom(_mm(vij, w('r4c')) * cmx + Vjl
                         + w('i_vb', True), Vik)
        P = p_s[...] * V
        p_s[...] = P
        r['xij_o'][0, 0] = P.astype(jnp.float32).reshape(A, NBH, NF)
        x_s[...] = x + vi

    @pl.when(i == NLAYER - 1)
    def _epilogue():
        r['xi_o'][0] = x_s[...]


def kernel(atomic_numbers, positions, cell, cell_offset, neighbors,
           neighbor_mask, params):
    del cell, cell_offset, neighbor_mask
    sch = params['schnet']
    fst = params['first']
    itr = params['inter']
    eye32 = np.eye(NCB, dtype=np.float32)

    def st(ps, lin, key):
        return jnp.stack([p[lin][key] for p in ps])

    arrs = {
        'az': atomic_numbers.astype(jnp.int32).reshape(B, 1, A),
        'pos': positions,
        'nbr': neighbors.astype(jnp.int32),
        'emb': params['emb'],
        'offs': jnp.asarray(_OFFS).reshape(1, N_GAUSS),
        'r4c': jnp.asarray(_R4),
        'r3c': jnp.asarray(_R3),
        't3c': jnp.asarray(_T3),
        's_f1w': st(sch, 'filt1', 'w'), 's_f1b': st(sch, 'filt1', 'b'),
        's_f2w': st(sch, 'filt2', 'w'), 's_f2b': st(sch, 'filt2', 'b'),
        's_inw': st(sch, 'in2f', 'w'),
        's_ow': st(sch, 'f2out', 'w'), 's_ob': st(sch, 'f2out', 'b'),
        's_dw': st(sch, 'dense', 'w'), 's_db': st(sch, 'dense', 'b'),
        'f_f1w': fst['filt1']['w'], 'f_f1b': fst['filt1']['b'],
        'f_f2w': fst['filt2']['w'], 'f_f2b': fst['filt2']['b'],
        'f_inw': fst['ft_in2f']['w'],
        'f_ftw': fst['ft_f2out']['w'], 'f_ftb': fst['ft_f2out']['b'],
        'f_a1w': fst['atom1']['w'], 'f_a1b': fst['atom1']['b'],
        'f_a2w': fst['atom2']['w'], 'f_a2b': fst['atom2']['b'],
        'f_p1w': fst['pair1']['w'], 'f_p1b': fst['pair1']['b'],
        'f_p2w': fst['pair2']['w'], 'f_p2b': fst['pair2']['b'],
        'f_e1w': fst['env1']['w'], 'f_e1b': fst['env1']['b'],
        'f_e2w': fst['env2']['w'], 'f_e2b': fst['env2']['b'],
        'i_f1w': st(itr, 'filt1', 'w'), 'i_f1b': st(itr, 'filt1', 'b'),
        'i_f2w': st(itr, 'filt2', 'w'), 'i_f2b': st(itr, 'filt2', 'b'),
        'i_inw': st(itr, 'ft_in2f', 'w'),
        'i_ftw': st(itr, 'ft_f2out', 'w'), 'i_ftb': st(itr, 'ft_f2out', 'b'),
        'i_a1w': st(itr, 'atom1', 'w'), 'i_a1b': st(itr, 'atom1', 'b'),
        'i_a2w': st(itr, 'atom2', 'w'), 'i_a2b': st(itr, 'atom2', 'b'),
        'i_p1w': st(itr, 'pair1', 'w'), 'i_p1b': st(itr, 'pair1', 'b'),
        'i_p2w': st(itr, 'pair2', 'w'), 'i_p2b': st(itr, 'pair2', 'b'),
        'i_e1w': st(itr, 'env1', 'w'), 'i_e1b': st(itr, 'env1', 'b'),
        'i_e2w': st(itr, 'env2', 'w'), 'i_e2b': st(itr, 'env2', 'b'),
        'i_pmt': jnp.pad(
            jnp.tile(st(itr, 'pair_mult', 'w'), (1, 1, NCB)),
            ((0, 0), (0, 5), (0, 0))),
        'i_g1': jnp.stack([jnp.kron(eye32, p['env_mult1']['w'])
                           for p in itr]),
        'i_g2': jnp.stack([jnp.kron(eye32, p['env_mult2']['w'])
                           for p in itr]),
        'i_vb': jnp.stack([jnp.tile(p['pair_mult']['b'], NCB)
                           + jnp.tile(p['env_mult1']['b'], NCB)
                           + jnp.tile(p['env_mult2']['b'], NCB)
                           for p in itr]).reshape(N_DIR, 1, NF),
    }

    keep_f32 = {'az', 'pos', 'nbr', 'emb', 'offs'}
    for n in list(arrs):
        if n.startswith('i_') and arrs[n].ndim == 2:
            arrs[n] = arrs[n].reshape(N_DIR, 1, arrs[n].shape[-1])
        if n not in keep_f32:
            arrs[n] = arrs[n].astype(_BF)

    names = list(arrs.keys())
    batch_arrs = {'az', 'pos', 'nbr'}
    layer_arrs = {n for n in names if n.startswith('i_')}

    def spec_for(name):
        shape = arrs[name].shape
        nd = len(shape)
        if name in batch_arrs:
            return pl.BlockSpec((1,) + shape[1:],
                                lambda b, i: (b,) + (0,) * (nd - 1))
        if name in layer_arrs:
            return pl.BlockSpec(
                (1,) + shape[1:],
                lambda b, i: (jnp.maximum(i - 1, 0),) + (0,) * (nd - 1))
        return pl.BlockSpec(shape, lambda b, i, _n=nd: (0,) * _n)

    in_specs = [spec_for(n) for n in names]
    out_names = ['x0_o', 'xi_o', 'xij_o']
    scratch_names = ['oh_s', 'geo_s', 'g_s', 'x_s', 'p_s']
    out_shape = [
        jax.ShapeDtypeStruct((B, A, NCB), jnp.float32),
        jax.ShapeDtypeStruct((B, A, NCB), jnp.float32),
        jax.ShapeDtypeStruct((B, NLAYER, A, NBH, NF), jnp.float32),
    ]
    out_specs = [
        pl.BlockSpec((1, A, NCB), lambda b, i: (b, 0, 0)),
        pl.BlockSpec((1, A, NCB), lambda b, i: (b, 0, 0)),
        pl.BlockSpec((1, 1, A, NBH, NF), lambda b, i: (b, i, 0, 0, 0)),
    ]
    scratch_shapes = [
        pltpu.VMEM((RWS, A), _BF),
        pltpu.VMEM((RWS, 8), _BF),
        pltpu.VMEM((RWS, N_GAUSS), _BF),
        pltpu.VMEM((A, NCB), jnp.float32),
        pltpu.VMEM((RWS, NF), _BF),
    ]

    fn = pl.pallas_call(
        functools.partial(_body, names + out_names + scratch_names),
        grid=(B, NLAYER),
        in_specs=in_specs,
        out_specs=out_specs,
        out_shape=out_shape,
        scratch_shapes=scratch_shapes,
        compiler_params=pltpu.CompilerParams(
            dimension_semantics=('parallel', 'arbitrary')),
    )
    x0, xi, xij = fn(*[arrs[n] for n in names])
    return x0, xi, jnp.transpose(xij, (0, 2, 3, 1, 4))


# speedup vs baseline: 1.2460x; 1.2460x over previous
import functools
import math

import jax
import jax.numpy as jnp
import numpy as np
from jax.experimental import pallas as pl
from jax.experimental.pallas import tpu as pltpu

B, A, NBH = 4, 64, 48
CUTOFF = 10.0
N_GAUSS = 50
NF = 128
NCB = 32
DIRS = 4
LMAX = 4
N_SCHNET = 2
N_DIR = 2 * LMAX
NLAYER = 1 + N_DIR
RWS = A * NBH
MAX_Z = 100

_OFFS = np.linspace(0.0, CUTOFF, N_GAUSS).astype(np.float32)
_GCOEF = np.float32(-0.5 / (_OFFS[1] - _OFFS[0]) ** 2)
_LN2 = np.float32(math.log(2.0))

_R4 = np.kron(np.eye(NCB, dtype=np.float32), np.ones((1, 4), np.float32))
_R3 = np.kron(np.eye(NCB, dtype=np.float32), np.ones((1, 3), np.float32))
_T3 = np.zeros((8, 3 * NCB), np.float32)
for _d in range(3):
    _T3[_d, _d::3] = 1.0

_PREC = jax.lax.Precision.DEFAULT
_BF = jnp.bfloat16


def _mm(a, b, prec=_PREC):
    return jax.lax.dot_general(
        a, b, (((a.ndim - 1,), (0,)), ((), ())),
        precision=prec, preferred_element_type=jnp.float32)


def _b(x):
    return x.astype(_BF)


_INV_LN2 = float(1.0 / math.log(2.0))
_LN2F = float(math.log(2.0))


def _hlog(z):
    nz = -z
    t = jnp.exp2(jnp.minimum(z, nz))
    return jnp.maximum(nz, 0.0) + jnp.log2(1.0 + t)


def _body(names, *refs):
    r = dict(zip(names, refs))
    i = pl.program_id(1)

    def w(name, squeeze=False):
        ref = r[name]
        return ref[0] if squeeze else ref[...]

    oh_s = r['oh_s']
    geo_s = r['geo_s']
    g_s = r['g_s']
    x_s = r['x_s']
    p_s = r['p_s']

    def filters(f1w, f1b, f2w, f2b, g, cut):
        W = _mm(_b(_hlog(_mm(g, f1w) + f1b)), f2w) + f2b
        return W * cut

    def add_per_atom(v, y):
        f = v.shape[-1]
        return (v.reshape(A, NBH, f) + y[:, None, :]).reshape(RWS, f)

    def mul_per_atom(v, y):
        f = v.shape[-1]
        return (v.reshape(A, NBH, f) * y[:, None, :]).reshape(RWS, f)

    def nsum(v):
        return jnp.sum(v.reshape(A, NBH, v.shape[-1]), axis=1)

    def ft_and_heads(x, oh, g, cut, p1w, p1b, p2w, p2b, inw, ftw, ftb,
                     a1w, a1b, a2w, a2b, e1w, e1b, e2w, e2b,
                     fw1, fb1, fw2, fb2):
        W = filters(fw1, fb1, fw2, fb2, g, cut)
        y = _mm(_b(x), inw)
        v = _hlog(_mm(_b(mul_per_atom(_mm(oh, _b(y)), y) * W), ftw) + ftb)
        vb = _b(v)
        vi = _mm(_b(_hlog(_mm(_b(nsum(v)), a1w) + a1b)), a2w) + a2b
        vij = _mm(_b(_hlog(_mm(vb, p1w) + p1b)), p2w) + p2b
        vik = _mm(_b(_hlog(_mm(vb, e1w) + e1b)), e2w) + e2b
        return vi, vij, vik

    @pl.when(i == 0)
    def _prologue():
        az = r['az'][0, 0]
        zidx = jax.lax.broadcasted_iota(jnp.int32, (A, MAX_Z), 1)
        ohz = (az[:, None] == zidx).astype(jnp.float32)
        x0 = _mm(ohz, w('emb'), jax.lax.Precision.HIGHEST)
        r['x0_o'][0] = x0

        nb = r['nbr'][0]
        jidx = jax.lax.broadcasted_iota(jnp.int32, (A, NBH, A), 2)
        ohf = (nb[:, :, None] == jidx).astype(jnp.float32).reshape(RWS, A)
        oh = ohf.astype(_BF)
        oh_s[...] = oh

        pos = r['pos'][0]
        pos_j = _mm(ohf, pos)
        d = (pos_j.reshape(A, NBH, 3) - pos[:, None, :]).reshape(RWS, 3)
        s = jnp.sum(d * d, axis=1, keepdims=True)
        rij = jnp.sqrt(jnp.where(s > 0, s, 1.0)) * (s > 0).astype(jnp.float32)
        cos = d / jnp.where(rij > 0, rij, 1.0)
        cut = (rij <= CUTOFF).astype(jnp.float32)
        geo_s[...] = jnp.concatenate(
            [cos, cut, jnp.zeros((RWS, 4), jnp.float32)], axis=1).astype(_BF)
        g = jnp.exp(_GCOEF * (rij - w('offs')) ** 2).astype(_BF)
        g_s[...] = g

        x = x0
        for k in range(N_SCHNET):
            W = filters(w('s_f1w')[k], w('s_f1b')[k], w('s_f2w')[k],
                        w('s_f2b')[k], g, cut)
            y_j = _mm(oh, _b(_mm(_b(x), w('s_inw')[k])))
            ysum = nsum(y_j * W)
            y2 = _hlog(_mm(_b(ysum), w('s_ow')[k]) + w('s_ob')[k])
            x = x + _mm(_b(y2), w('s_dw')[k]) + w('s_db')[k]

        vi, vij, vik = ft_and_heads(
            x, oh, g, cut,
            w('f_p1w'), w('f_p1b'), w('f_p2w'), w('f_p2b'),
            w('f_inw'), w('f_ftw'), w('f_ftb'),
            w('f_a1w'), w('f_a1b'), w('f_a2w'), w('f_a2b'),
            w('f_e1w'), w('f_e1b'), w('f_e2w'), w('f_e2b'),
            w('f_f1w'), w('f_f1b'), w('f_f2w'), w('f_f2b'))
        vik32 = nsum(vik)
        v32 = add_per_atom(vij + _mm(oh, _b(vik32)), vik32)
        P = _mm(_b(v32), w('r4c'))
        p_s[...] = P
        r['xij_o'][0, 0] = P.reshape(A, NBH, NF)
        x_s[...] = x + vi

    @pl.when(i > 0)
    def _direction_layer():
        oh = oh_s[...]
        geo = geo_s[...]
        g = g_s[...]
        cut = geo[:, 3:4].astype(jnp.float32)
        x = x_s[...]

        vi, vij, vik = ft_and_heads(
            x, oh, g, cut,
            w('i_p1w', True), w('i_p1b', True), w('i_p2w', True),
            w('i_p2b', True), w('i_inw', True), w('i_ftw', True),
            w('i_ftb', True), w('i_a1w', True), w('i_a1b', True),
            w('i_a2w', True), w('i_a2b', True), w('i_e1w', True),
            w('i_e1b', True), w('i_e2w', True), w('i_e2b', True),
            w('i_f1w', True), w('i_f1b', True), w('i_f2w', True),
            w('i_f2b', True))

        cosx96 = _mm(geo, w('t3c'))
        vik96 = _mm(_b(vik), w('r3c')) * cosx96
        Vik96 = _b(nsum(vik96))
        Vjl = _mm(oh, _b(_mm(Vik96, w('i_g2', True))))
        Vik = _mm(Vik96, w('i_g1', True))
        cmx = _mm(geo, w('i_pmt', True))
        V = add_per_atom(_mm(_b(vij), w('r4c')) * cmx + Vjl
                         + w('i_vb', True), Vik)
        P = p_s[...] * V
        p_s[...] = P
        r['xij_o'][0, 0] = P.reshape(A, NBH, NF)
        x_s[...] = x + vi

    @pl.when(i == NLAYER - 1)
    def _epilogue():
        r['xi_o'][0] = x_s[...]


def kernel(atomic_numbers, positions, cell, cell_offset, neighbors,
           neighbor_mask, params):
    del cell, cell_offset, neighbor_mask
    sch = params['schnet']
    fst = params['first']
    itr = params['inter']
    eye32 = np.eye(NCB, dtype=np.float32)

    def st(ps, lin, key):
        return jnp.stack([p[lin][key] for p in ps])

    arrs = {
        'az': atomic_numbers.astype(jnp.int32).reshape(B, 1, A),
        'pos': positions,
        'nbr': neighbors.astype(jnp.int32),
        'emb': params['emb'],
        'offs': jnp.asarray(_OFFS).reshape(1, N_GAUSS),
        'r4c': jnp.asarray(_R4),
        'r3c': jnp.asarray(_R3),
        't3c': jnp.asarray(_T3),
        's_f1w': st(sch, 'filt1', 'w'), 's_f1b': st(sch, 'filt1', 'b'),
        's_f2w': st(sch, 'filt2', 'w'), 's_f2b': st(sch, 'filt2', 'b'),
        's_inw': st(sch, 'in2f', 'w'),
        's_ow': st(sch, 'f2out', 'w'), 's_ob': st(sch, 'f2out', 'b'),
        's_dw': st(sch, 'dense', 'w'), 's_db': st(sch, 'dense', 'b'),
        'f_f1w': fst['filt1']['w'], 'f_f1b': fst['filt1']['b'],
        'f_f2w': fst['filt2']['w'], 'f_f2b': fst['filt2']['b'],
        'f_inw': fst['ft_in2f']['w'],
        'f_ftw': fst['ft_f2out']['w'], 'f_ftb': fst['ft_f2out']['b'],
        'f_a1w': fst['atom1']['w'], 'f_a1b': fst['atom1']['b'],
        'f_a2w': fst['atom2']['w'], 'f_a2b': fst['atom2']['b'],
        'f_p1w': fst['pair1']['w'], 'f_p1b': fst['pair1']['b'],
        'f_p2w': fst['pair2']['w'], 'f_p2b': fst['pair2']['b'],
        'f_e1w': fst['env1']['w'], 'f_e1b': fst['env1']['b'],
        'f_e2w': fst['env2']['w'], 'f_e2b': fst['env2']['b'],
        'i_f1w': st(itr, 'filt1', 'w'), 'i_f1b': st(itr, 'filt1', 'b'),
        'i_f2w': st(itr, 'filt2', 'w'), 'i_f2b': st(itr, 'filt2', 'b'),
        'i_inw': st(itr, 'ft_in2f', 'w'),
        'i_ftw': st(itr, 'ft_f2out', 'w'), 'i_ftb': st(itr, 'ft_f2out', 'b'),
        'i_a1w': st(itr, 'atom1', 'w'), 'i_a1b': st(itr, 'atom1', 'b'),
        'i_a2w': st(itr, 'atom2', 'w'), 'i_a2b': st(itr, 'atom2', 'b'),
        'i_p1w': st(itr, 'pair1', 'w'), 'i_p1b': st(itr, 'pair1', 'b'),
        'i_p2w': st(itr, 'pair2', 'w'), 'i_p2b': st(itr, 'pair2', 'b'),
        'i_e1w': st(itr, 'env1', 'w'), 'i_e1b': st(itr, 'env1', 'b'),
        'i_e2w': st(itr, 'env2', 'w'), 'i_e2b': st(itr, 'env2', 'b'),
        'i_pmt': jnp.pad(
            jnp.tile(st(itr, 'pair_mult', 'w'), (1, 1, NCB)),
            ((0, 0), (0, 5), (0, 0))),
        'i_g1': jnp.stack([jnp.kron(eye32, p['env_mult1']['w'])
                           for p in itr]),
        'i_g2': jnp.stack([jnp.kron(eye32, p['env_mult2']['w'])
                           for p in itr]),
        'i_vb': jnp.stack([jnp.tile(p['pair_mult']['b'], NCB)
                           + jnp.tile(p['env_mult1']['b'], NCB)
                           + jnp.tile(p['env_mult2']['b'], NCB)
                           for p in itr]).reshape(N_DIR, 1, NF),
    }

    def _fold(pw, pb, consumers):
        for cw, cb, mult in consumers:
            Wc = arrs[cw]
            arrs[cb] = arrs[cb] - (mult * _LN2) * Wc.sum(axis=-2)
            arrs[cw] = Wc * _LN2
        arrs[pw] = arrs[pw] * np.float32(-_INV_LN2)
        arrs[pb] = arrs[pb] * np.float32(-_INV_LN2)

    _fold('s_f1w', 's_f1b', [('s_f2w', 's_f2b', 1.0)])
    _fold('s_ow', 's_ob', [('s_dw', 's_db', 1.0)])
    for p in ('f_', 'i_'):
        _fold(p + 'f1w', p + 'f1b', [(p + 'f2w', p + 'f2b', 1.0)])
        _fold(p + 'ftw', p + 'ftb', [(p + 'p1w', p + 'p1b', 1.0),
                                     (p + 'e1w', p + 'e1b', 1.0),
                                     (p + 'a1w', p + 'a1b', float(NBH))])
        _fold(p + 'a1w', p + 'a1b', [(p + 'a2w', p + 'a2b', 1.0)])
        _fold(p + 'p1w', p + 'p1b', [(p + 'p2w', p + 'p2b', 1.0)])
        _fold(p + 'e1w', p + 'e1b', [(p + 'e2w', p + 'e2b', 1.0)])

    keep_f32 = {'az', 'pos', 'nbr', 'emb', 'offs', 'i_vb'}
    for n in list(arrs):
        if n.startswith('i_') and arrs[n].ndim == 2:
            arrs[n] = arrs[n].reshape(N_DIR, 1, arrs[n].shape[-1])
        if n in keep_f32:
            continue
        if n.endswith('b'):
            arrs[n] = arrs[n].astype(jnp.float32)
        else:
            arrs[n] = arrs[n].astype(_BF)

    names = list(arrs.keys())
    batch_arrs = {'az', 'pos', 'nbr'}
    layer_arrs = {n for n in names if n.startswith('i_')}

    def spec_for(name):
        shape = arrs[name].shape
        nd = len(shape)
        if name in batch_arrs:
            return pl.BlockSpec((1,) + shape[1:],
                                lambda b, i: (b,) + (0,) * (nd - 1))
        if name in layer_arrs:
            return pl.BlockSpec(
                (1,) + shape[1:],
                lambda b, i: (jnp.maximum(i - 1, 0),) + (0,) * (nd - 1))
        return pl.BlockSpec(shape, lambda b, i, _n=nd: (0,) * _n)

    in_specs = [spec_for(n) for n in names]
    out_names = ['x0_o', 'xi_o', 'xij_o']
    scratch_names = ['oh_s', 'geo_s', 'g_s', 'x_s', 'p_s']
    out_shape = [
        jax.ShapeDtypeStruct((B, A, NCB), jnp.float32),
        jax.ShapeDtypeStruct((B, A, NCB), jnp.float32),
        jax.ShapeDtypeStruct((B, NLAYER, A, NBH, NF), jnp.float32),
    ]
    out_specs = [
        pl.BlockSpec((1, A, NCB), lambda b, i: (b, 0, 0)),
        pl.BlockSpec((1, A, NCB), lambda b, i: (b, 0, 0)),
        pl.BlockSpec((1, 1, A, NBH, NF), lambda b, i: (b, i, 0, 0, 0)),
    ]
    scratch_shapes = [
        pltpu.VMEM((RWS, A), _BF),
        pltpu.VMEM((RWS, 8), _BF),
        pltpu.VMEM((RWS, N_GAUSS), _BF),
        pltpu.VMEM((A, NCB), jnp.float32),
        pltpu.VMEM((RWS, NF), jnp.float32),
    ]

    fn = pl.pallas_call(
        functools.partial(_body, names + out_names + scratch_names),
        grid=(B, NLAYER),
        in_specs=in_specs,
        out_specs=out_specs,
        out_shape=out_shape,
        scratch_shapes=scratch_shapes,
        compiler_params=pltpu.CompilerParams(
            dimension_semantics=('parallel', 'arbitrary')),
    )
    x0, xi, xij = fn(*[arrs[n] for n in names])
    return x0, xi, jnp.transpose(xij, (0, 2, 3, 1, 4))
